# baseline (speedup 1.0000x reference)
import jax
import jax.numpy as jnp
from jax.experimental import pallas as pl
from jax.experimental.pallas import tpu as pltpu

_BL = 256


def _ln_kernel(x_ref, t_ref, g_ref, b_ref, o_ref):
    h = x_ref[...] + t_ref[None]
    mean = jnp.mean(h, axis=2, keepdims=True)
    c = h - mean
    var = jnp.mean(c * c, axis=2, keepdims=True)
    o_ref[...] = (c * jax.lax.rsqrt(var + 1e-5)) * g_ref[...] + b_ref[...]


def kernel(x, table, gamma, beta):
    b, l, h = x.shape
    grid = (l // _BL,)
    return pl.pallas_call(
        _ln_kernel,
        grid=grid,
        in_specs=[
            pl.BlockSpec((b, _BL, h), lambda i: (0, i, 0)),
            pl.BlockSpec((_BL, h), lambda i: (i, 0)),
            pl.BlockSpec((1, h), lambda i: (0, 0)),
            pl.BlockSpec((1, h), lambda i: (0, 0)),
        ],
        out_specs=pl.BlockSpec((b, _BL, h), lambda i: (0, i, 0)),
        out_shape=jax.ShapeDtypeStruct((b, l, h), x.dtype),
        compiler_params=pltpu.CompilerParams(
            dimension_semantics=("arbitrary",),
        ),
    )(x, table, gamma.reshape(1, h), beta.reshape(1, h))


# speedup vs baseline: 1.0204x; 1.0204x over previous
import jax
import jax.numpy as jnp
from jax.experimental import pallas as pl
from jax.experimental.pallas import tpu as pltpu

_BL = 512


def _ln_kernel(x_ref, t_ref, g_ref, b_ref, o_ref):
    h = x_ref[...] + t_ref[None]
    mean = jnp.mean(h, axis=2, keepdims=True)
    c = h - mean
    var = jnp.mean(c * c, axis=2, keepdims=True)
    o_ref[...] = (c * jax.lax.rsqrt(var + 1e-5)) * g_ref[...] + b_ref[...]


def kernel(x, table, gamma, beta):
    b, l, h = x.shape
    grid = (l // _BL,)
    return pl.pallas_call(
        _ln_kernel,
        grid=grid,
        in_specs=[
            pl.BlockSpec((b, _BL, h), lambda i: (0, i, 0)),
            pl.BlockSpec((_BL, h), lambda i: (i, 0)),
            pl.BlockSpec((1, h), lambda i: (0, 0)),
            pl.BlockSpec((1, h), lambda i: (0, 0)),
        ],
        out_specs=pl.BlockSpec((b, _BL, h), lambda i: (0, i, 0)),
        out_shape=jax.ShapeDtypeStruct((b, l, h), x.dtype),
        compiler_params=pltpu.CompilerParams(
            dimension_semantics=("arbitrary",),
        ),
    )(x, table, gamma.reshape(1, h), beta.reshape(1, h))
